# baseline (speedup 1.0000x reference)
import functools

import jax
import jax.numpy as jnp
from jax import lax
from jax.experimental import pallas as pl
from jax.experimental.pallas import tpu as pltpu
from jax.experimental.pallas import tpu_sc as plsc

B = 16384
D = 32
NC = 2
NS = 16
NW = NC * NS
B_PER_W = B // NW
CHUNK = 128
NCHUNK = B_PER_W // CHUNK

_mesh = plsc.VectorSubcoreMesh(core_axis_name="c", subcore_axis_name="s")


@functools.partial(
    pl.kernel,
    mesh=_mesh,
    out_type=jax.ShapeDtypeStruct((B, D), jnp.float32),
    compiler_params=pltpu.CompilerParams(use_tc_tiling_on_sc=False),
    scratch_types=[
        pltpu.VMEM((NCHUNK, CHUNK), jnp.int32),
        pltpu.VMEM((B_PER_W, D), jnp.float32),
        pltpu.SemaphoreType.DMA,
    ],
)
def _embed_gather(labels_hbm, table_hbm, out_hbm, idx_v, rows_v, sem):
    wid = lax.axis_index("s") * NC + lax.axis_index("c")
    base = wid * B_PER_W
    for j in range(NCHUNK):
        pltpu.sync_copy(
            labels_hbm.at[pl.ds(base + j * CHUNK, CHUNK)],
            idx_v.at[j],
        )
    copies = [
        pltpu.async_copy(
            table_hbm.at[idx_v.at[j]],
            rows_v.at[pl.ds(j * CHUNK, CHUNK)],
            sem,
        )
        for j in range(NCHUNK)
    ]
    for c in copies:
        c.wait()
    pltpu.sync_copy(rows_v, out_hbm.at[pl.ds(base, B_PER_W)])


def kernel(labels, train, table):
    del train
    return _embed_gather(labels.astype(jnp.int32), table)


# speedup vs baseline: 2.2040x; 2.2040x over previous
import functools

import jax
import jax.numpy as jnp
from jax import lax
from jax.experimental import pallas as pl
from jax.experimental.pallas import tpu as pltpu
from jax.experimental.pallas import tpu_sc as plsc

B = 16384
D = 32
R = 8
V = 1_000_000
NSLAB = V // R
NC = 2
NS = 16
NW = NC * NS
B_PER_W = B // NW
K = 16

_mesh = plsc.VectorSubcoreMesh(core_axis_name="c", subcore_axis_name="s")


@functools.partial(
    pl.kernel,
    mesh=_mesh,
    out_type=jax.ShapeDtypeStruct((B, D), jnp.float32),
    compiler_params=pltpu.CompilerParams(needs_layout_passes=False),
    scratch_types=[
        pltpu.VMEM((B_PER_W,), jnp.int32),
        pltpu.VMEM((K, R, D), jnp.float32),
        pltpu.VMEM((B_PER_W, D), jnp.float32),
        [pltpu.SemaphoreType.DMA] * K,
    ],
)
def _embed_gather(labels_hbm, table3_hbm, out_hbm,
                  lab_v, ring, out_v, sems):
    wid = lax.axis_index("s") * NC + lax.axis_index("c")
    base = wid * B_PER_W

    pltpu.sync_copy(labels_hbm.at[pl.ds(base, B_PER_W)], lab_v)

    def fire(lab_i, b):
        t = lax.shift_right_logical(lab_i, 3)
        pltpu.async_copy(table3_hbm.at[t], ring.at[b], sems[b])

    def extract(lab_i, i, b):
        pltpu.make_async_copy(table3_hbm.at[0], ring.at[b], sems[b]).wait()
        j = jnp.bitwise_and(lab_i, 7)
        out_v[i, pl.ds(0, 16)] = ring[b, j, pl.ds(0, 16)]
        out_v[i, pl.ds(16, 16)] = ring[b, j, pl.ds(16, 16)]

    lab0 = lab_v[pl.ds(0, K)]
    for b in range(K):
        fire(lab0[b], b)

    nchunk = B_PER_W // K

    def body(chunk, _):
        i0 = chunk * K
        cur = lab_v[pl.ds(i0, K)]
        nxt = lab_v[pl.ds(jnp.minimum(i0 + K, B_PER_W - K), K)]
        last = chunk == nchunk - 1
        for b in range(K):
            extract(cur[b], i0 + b, b)

            @pl.when(jnp.logical_not(last))
            def _():
                fire(nxt[b], b)

        return 0

    lax.fori_loop(0, nchunk, body, 0)

    pltpu.sync_copy(out_v, out_hbm.at[pl.ds(base, B_PER_W)])


def kernel(labels, train, table):
    del train
    table3 = table.reshape(NSLAB, R, D)
    return _embed_gather(labels.astype(jnp.int32), table3)


# speedup vs baseline: 2.7681x; 1.2559x over previous
import functools

import jax
import jax.numpy as jnp
from jax import lax
from jax.experimental import pallas as pl
from jax.experimental.pallas import tpu as pltpu
from jax.experimental.pallas import tpu_sc as plsc

B = 16384
D = 32
R = 8
V = 1_000_000
NSLAB = V // R
NC = 2
NS = 16
NW = NC * NS
B_PER_W = B // NW
G = 16
NGRP = B_PER_W // G

_mesh = plsc.VectorSubcoreMesh(core_axis_name="c", subcore_axis_name="s")


@functools.partial(
    pl.kernel,
    mesh=_mesh,
    out_type=jax.ShapeDtypeStruct((B, D), jnp.float32),
    compiler_params=pltpu.CompilerParams(needs_layout_passes=False),
    scratch_types=[
        pltpu.VMEM((B_PER_W,), jnp.int32),
        pltpu.VMEM((B_PER_W, D), jnp.float32),
        pltpu.SemaphoreType.DMA,
    ],
)
def _embed_gather(labels_hbm, table3_hbm, out_hbm, lab_v, out_v, sem):
    wid = lax.axis_index("s") * NC + lax.axis_index("c")
    base = wid * B_PER_W

    pltpu.sync_copy(labels_hbm.at[pl.ds(base, B_PER_W)], lab_v)

    def body(g, _):
        lab = lab_v[pl.ds(g * G, G)]
        for l in range(G):
            lab_i = lab[l]
            t = lax.shift_right_logical(lab_i, 3)
            j = jnp.bitwise_and(lab_i, 7)
            pltpu.async_copy(table3_hbm.at[t, j], out_v.at[g * G + l], sem)
        return 0

    lax.fori_loop(0, NGRP, body, 0)

    pltpu.make_async_copy(out_hbm.at[pl.ds(base, B_PER_W)], out_v, sem).wait()
    pltpu.sync_copy(out_v, out_hbm.at[pl.ds(base, B_PER_W)])


def kernel(labels, train, table):
    del train
    table3 = table.reshape(NSLAB, R, D)
    return _embed_gather(labels.astype(jnp.int32), table3)
